# baseline (speedup 1.0000x reference)
import jax
import jax.numpy as jnp
from jax.experimental import pallas as pl
from jax.experimental.pallas import tpu as pltpu


def _proj_body(u_ref, w_ref, b_ref, o_ref):
    acc = jnp.dot(u_ref[...], w_ref[...], preferred_element_type=jnp.float32)
    o_ref[...] = (acc + b_ref[...]).astype(o_ref.dtype)


def _project(ut, wx, bx, interpret=False):
    M, D = ut.shape
    N = wx.shape[1]
    bm = min(1024, M)
    grid = (M // bm,)
    return pl.pallas_call(
        _proj_body,
        out_shape=jax.ShapeDtypeStruct((M, N), jnp.bfloat16),
        grid=grid,
        in_specs=[
            pl.BlockSpec((bm, D), lambda i: (i, 0)),
            pl.BlockSpec((D, N), lambda i: (0, 0)),
            pl.BlockSpec((1, N), lambda i: (0, 0)),
        ],
        out_specs=pl.BlockSpec((bm, N), lambda i: (i, 0)),
        compiler_params=pltpu.CompilerParams(
            dimension_semantics=("parallel",),
            vmem_limit_bytes=48 * 1024 * 1024,
        ),
        name="speaker_rnn_project",
        interpret=interpret,
    )(ut, wx, bx)


def _sig(x):
    return 0.5 + 0.5 * jnp.tanh(0.5 * x)


def _make_rnn_body(H, unroll):
    f32 = jnp.float32
    bf16 = jnp.bfloat16

    def _rnn_body(spk_ref, xp_ref, wg_ref, wsg_ref, wsh_ref, wes_ref, bn_ref,
                  out_ref, g_ref, s0_ref, s1_ref):
        t = pl.program_id(0)

        @pl.when(t == 0)
        def _():
            g_ref[...] = jnp.zeros_like(g_ref)
            s0_ref[...] = jnp.zeros_like(s0_ref)
            s1_ref[...] = jnp.zeros_like(s1_ref)

        B = g_ref.shape[0]
        s_news = []

        def xs(u, c):
            return xp_ref[u * B:(u + 1) * B, c * H:(c + 1) * H].astype(f32)

        for u in range(unroll):
            g = g_ref[...]

            hh = jnp.dot(g.astype(bf16), wg_ref[...],
                         preferred_element_type=f32)
            r = _sig(xs(u, 0) + hh[:, :H])
            z = _sig(xs(u, 1) + hh[:, H:2 * H])
            n = jnp.tanh(xs(u, 2) + r * (hh[:, 2 * H:] + bn_ref[0:1, :]))
            g_new = (1.0 - z) * n + z * g
            g_ref[...] = g_new

            m = jnp.transpose(spk_ref[0, u:u + 1, :], (1, 0))
            s0 = s0_ref[...]
            s1 = s1_ref[...]
            s_prev = jnp.where(m < 0.5, s0, s1)
            sg = jnp.dot(g_new.astype(bf16), wsg_ref[...],
                         preferred_element_type=f32)
            sh = jnp.dot(s_prev.astype(bf16), wsh_ref[...],
                         preferred_element_type=f32)
            r_s = _sig(xs(u, 3) + sg[:, :H] + sh[:, :H])
            z_s = _sig(xs(u, 4) + sg[:, H:2 * H] + sh[:, H:2 * H])
            n_s = jnp.tanh(xs(u, 5) + sg[:, 2 * H:]
                           + r_s * (sh[:, 2 * H:] + bn_ref[1:2, :]))
            s_new = (1.0 - z_s) * n_s + z_s * s_prev
            s0_ref[...] = jnp.where(m < 0.5, s_new, s0)
            s1_ref[...] = jnp.where(m < 0.5, s1, s_new)
            s_news.append(s_new.astype(bf16))

        s_cat = jnp.concatenate(s_news, axis=0)
        es = jnp.dot(s_cat, wes_ref[...], preferred_element_type=f32)
        pex = jnp.concatenate(
            [xp_ref[u * B:(u + 1) * B, 6 * H:] for u in range(unroll)],
            axis=0).astype(f32)
        pe = pex + es
        r_e = _sig(pe[:, :H])
        z_e = _sig(pe[:, H:2 * H])
        n_e = jnp.tanh(pe[:, 2 * H:] + r_e * bn_ref[2:3, :])
        out_ref[...] = ((1.0 - z_e) * n_e).astype(out_ref.dtype)

    return _rnn_body


def _forward(utt_embeds, speaker_ids,
             gW_ih, gW_hh, gb_ih, gb_hh,
             sW_ih, sW_hh, sb_ih, sb_hh,
             eW_ih, eW_hh, eb_ih, eb_hh,
             interpret=False):
    B, T, D = utt_embeds.shape
    H = gW_hh.shape[1]

    f32 = jnp.float32
    bf16 = jnp.bfloat16

    wx = jnp.concatenate([gW_ih, sW_ih[:, :D], eW_ih[:, :D]], axis=0).T

    def fold(b_ih, b_hh):
        return b_ih + jnp.concatenate([b_hh[:2 * H], jnp.zeros((H,), f32)])

    bx = jnp.concatenate(
        [fold(gb_ih, gb_hh), fold(sb_ih, sb_hh), fold(eb_ih, eb_hh)]
    ).reshape(1, 9 * H).astype(f32)

    ut = jnp.swapaxes(utt_embeds.astype(bf16), 0, 1).reshape(T * B, D)
    xp = _project(ut, wx.astype(bf16), bx, interpret=interpret)

    UNROLL = 4
    spk = jnp.swapaxes(speaker_ids, 0, 1).astype(f32).reshape(
        T // UNROLL, UNROLL, B)

    wg = gW_hh.T.astype(bf16)
    wsg = sW_ih[:, D:].T.astype(bf16)
    wsh = sW_hh.T.astype(bf16)
    wes = eW_ih[:, D:].T.astype(bf16)
    bn = jnp.stack([gb_hh[2 * H:], sb_hh[2 * H:], eb_hh[2 * H:]]).astype(f32)

    out = pl.pallas_call(
        _make_rnn_body(H, UNROLL),
        out_shape=jax.ShapeDtypeStruct((T * B, H), jnp.bfloat16),
        grid=(T // UNROLL,),
        in_specs=[
            pl.BlockSpec((1, UNROLL, B), lambda t: (t, 0, 0)),
            pl.BlockSpec((UNROLL * B, 9 * H), lambda t: (t, 0)),
            pl.BlockSpec((H, 3 * H), lambda t: (0, 0)),
            pl.BlockSpec((H, 3 * H), lambda t: (0, 0)),
            pl.BlockSpec((H, 3 * H), lambda t: (0, 0)),
            pl.BlockSpec((H, 3 * H), lambda t: (0, 0)),
            pl.BlockSpec((3, H), lambda t: (0, 0)),
        ],
        out_specs=pl.BlockSpec((UNROLL * B, H), lambda t: (t, 0)),
        scratch_shapes=[
            pltpu.VMEM((B, H), jnp.float32),
            pltpu.VMEM((B, H), jnp.float32),
            pltpu.VMEM((B, H), jnp.float32),
        ],
        compiler_params=pltpu.CompilerParams(
            dimension_semantics=("arbitrary",),
            vmem_limit_bytes=48 * 1024 * 1024,
        ),
        name="speaker_rnn_recurrence",
        interpret=interpret,
    )(spk, xp, wg, wsg, wsh, wes, bn)

    return jnp.swapaxes(out.reshape(T, B, H), 0, 1).astype(jnp.float32)


def kernel(utt_embeds, speaker_ids,
           gW_ih, gW_hh, gb_ih, gb_hh,
           sW_ih, sW_hh, sb_ih, sb_hh,
           eW_ih, eW_hh, eb_ih, eb_hh):
    return _forward(utt_embeds, speaker_ids,
                    gW_ih, gW_hh, gb_ih, gb_hh,
                    sW_ih, sW_hh, sb_ih, sb_hh,
                    eW_ih, eW_hh, eb_ih, eb_hh)


# speedup vs baseline: 1.0371x; 1.0371x over previous
import jax
import jax.numpy as jnp
from jax.experimental import pallas as pl
from jax.experimental.pallas import tpu as pltpu


def _sig(x):
    return 0.5 + 0.5 * jnp.tanh(0.5 * x)


def _make_body(H, unroll, nblocks):
    f32 = jnp.float32
    bf16 = jnp.bfloat16

    def _body(spk_ref, utn_ref, wx_ref, bx_ref,
              wg_ref, wsg_ref, wsh_ref, wes_ref, bn_ref,
              out_ref, g_ref, s0_ref, s1_ref, xp_scr):
        t = pl.program_id(0)

        @pl.when(t == 0)
        def _():
            g_ref[...] = jnp.zeros_like(g_ref)
            s0_ref[...] = jnp.zeros_like(s0_ref)
            s1_ref[...] = jnp.zeros_like(s1_ref)

        x_in = utn_ref[...]
        for c3 in range(3):
            lo, hi = c3 * 3 * H, (c3 + 1) * 3 * H
            acc = jnp.dot(x_in, wx_ref[:, lo:hi], preferred_element_type=f32)
            xp_scr[:, lo:hi] = (acc + bx_ref[:, lo:hi]).astype(bf16)
        xv = xp_scr
        B = g_ref.shape[0]
        s_news = []

        def xs(u, c):
            return xv[u * B:(u + 1) * B, c * H:(c + 1) * H].astype(f32)

        for u in range(unroll):
            g = g_ref[...]

            hh = jnp.dot(g.astype(bf16), wg_ref[...],
                         preferred_element_type=f32)
            r = _sig(xs(u, 0) + hh[:, :H])
            z = _sig(xs(u, 1) + hh[:, H:2 * H])
            n = jnp.tanh(xs(u, 2) + r * (hh[:, 2 * H:] + bn_ref[0:1, :]))
            g_new = (1.0 - z) * n + z * g
            g_ref[...] = g_new

            m = jnp.transpose(spk_ref[0, u:u + 1, :], (1, 0))
            s0 = s0_ref[...]
            s1 = s1_ref[...]
            s_prev = jnp.where(m < 0.5, s0, s1)
            sg = jnp.dot(g_new.astype(bf16), wsg_ref[...],
                         preferred_element_type=f32)
            sh = jnp.dot(s_prev.astype(bf16), wsh_ref[...],
                         preferred_element_type=f32)
            r_s = _sig(xs(u, 3) + sg[:, :H] + sh[:, :H])
            z_s = _sig(xs(u, 4) + sg[:, H:2 * H] + sh[:, H:2 * H])
            n_s = jnp.tanh(xs(u, 5) + sg[:, 2 * H:]
                           + r_s * (sh[:, 2 * H:] + bn_ref[1:2, :]))
            s_new = (1.0 - z_s) * n_s + z_s * s_prev
            s0_ref[...] = jnp.where(m < 0.5, s_new, s0)
            s1_ref[...] = jnp.where(m < 0.5, s1, s_new)
            s_news.append(s_new.astype(bf16))

        s_cat = jnp.concatenate(s_news, axis=0)
        es = jnp.dot(s_cat, wes_ref[...], preferred_element_type=f32)
        pex = jnp.concatenate(
            [xv[u * B:(u + 1) * B, 6 * H:] for u in range(unroll)],
            axis=0).astype(f32)
        pe = pex + es
        r_e = _sig(pe[:, :H])
        z_e = _sig(pe[:, H:2 * H])
        n_e = jnp.tanh(pe[:, 2 * H:] + r_e * bn_ref[2:3, :])
        out_ref[...] = (1.0 - z_e) * n_e

    return _body


def _forward(utt_embeds, speaker_ids,
             gW_ih, gW_hh, gb_ih, gb_hh,
             sW_ih, sW_hh, sb_ih, sb_hh,
             eW_ih, eW_hh, eb_ih, eb_hh,
             interpret=False):
    B, T, D = utt_embeds.shape
    H = gW_hh.shape[1]

    f32 = jnp.float32
    bf16 = jnp.bfloat16

    wx = jnp.concatenate([gW_ih, sW_ih[:, :D], eW_ih[:, :D]], axis=0).T

    def fold(b_ih, b_hh):
        return b_ih + jnp.concatenate([b_hh[:2 * H], jnp.zeros((H,), f32)])

    bx = jnp.concatenate(
        [fold(gb_ih, gb_hh), fold(sb_ih, sb_hh), fold(eb_ih, eb_hh)]
    ).reshape(1, 9 * H).astype(f32)

    ut = jnp.swapaxes(utt_embeds.astype(bf16), 0, 1).reshape(T * B, D)

    UNROLL = 4
    NB = T // UNROLL
    spk = jnp.swapaxes(speaker_ids, 0, 1).astype(f32).reshape(NB, UNROLL, B)

    wg = gW_hh.T.astype(bf16)
    wsg = sW_ih[:, D:].T.astype(bf16)
    wsh = sW_hh.T.astype(bf16)
    wes = eW_ih[:, D:].T.astype(bf16)
    bn = jnp.stack([gb_hh[2 * H:], sb_hh[2 * H:], eb_hh[2 * H:]]).astype(f32)

    UB = UNROLL * B
    out = pl.pallas_call(
        _make_body(H, UNROLL, NB),
        out_shape=jax.ShapeDtypeStruct((T * B, H), jnp.float32),
        grid=(NB,),
        in_specs=[
            pl.BlockSpec((1, UNROLL, B), lambda t: (t, 0, 0)),
            pl.BlockSpec((UB, D), lambda t: (t, 0)),
            pl.BlockSpec((D, 9 * H), lambda t: (0, 0)),
            pl.BlockSpec((1, 9 * H), lambda t: (0, 0)),
            pl.BlockSpec((H, 3 * H), lambda t: (0, 0)),
            pl.BlockSpec((H, 3 * H), lambda t: (0, 0)),
            pl.BlockSpec((H, 3 * H), lambda t: (0, 0)),
            pl.BlockSpec((H, 3 * H), lambda t: (0, 0)),
            pl.BlockSpec((3, H), lambda t: (0, 0)),
        ],
        out_specs=pl.BlockSpec((UB, H), lambda t: (t, 0)),
        scratch_shapes=[
            pltpu.VMEM((B, H), jnp.float32),
            pltpu.VMEM((B, H), jnp.float32),
            pltpu.VMEM((B, H), jnp.float32),
            pltpu.VMEM((UB, 9 * H), jnp.bfloat16),
        ],
        compiler_params=pltpu.CompilerParams(
            dimension_semantics=("arbitrary",),
            vmem_limit_bytes=48 * 1024 * 1024,
        ),
        name="speaker_rnn_fused",
        interpret=interpret,
    )(spk, ut, wx.astype(bf16), bx, wg, wsg, wsh, wes, bn)

    return jnp.swapaxes(out.reshape(T, B, H), 0, 1)


def kernel(utt_embeds, speaker_ids,
           gW_ih, gW_hh, gb_ih, gb_hh,
           sW_ih, sW_hh, sb_ih, sb_hh,
           eW_ih, eW_hh, eb_ih, eb_hh):
    return _forward(utt_embeds, speaker_ids,
                    gW_ih, gW_hh, gb_ih, gb_hh,
                    sW_ih, sW_hh, sb_ih, sb_hh,
                    eW_ih, eW_hh, eb_ih, eb_hh)
